# baseline (speedup 1.0000x reference)
import functools

import jax
import jax.numpy as jnp
from jax import lax
from jax.experimental import pallas as pl
from jax.experimental.pallas import tpu as pltpu
from jax.experimental.pallas import tpu_sc as plsc

NC = 2
NS = 16


def _pre_body(x_ref, wx_ref, ws_ref, b_ref, ccle_ref, wc_ref, nd_ref, rel_ref,
              yt_ref, z_ref, gi_ref):
    dn = (((1,), (1,)), ((), ()))
    xv = x_ref[...]
    y = lax.dot_general(xv, wx_ref[...], dn,
                        preferred_element_type=jnp.float32)
    t4 = lax.dot_general(ccle_ref[...], wc_ref[...], dn,
                         preferred_element_type=jnp.float32)
    B, OUT = y.shape
    R = t4.shape[0]
    yt = (y[:, None, :] + t4[None, :, :]).reshape(B * R, OUT)
    bits = lax.bitcast_convert_type(yt, jnp.uint32)
    rnd = (bits + 0x7FFF + ((bits >> 16) & 1)) >> 16
    packed = rnd[:, : OUT // 2] | (rnd[:, OUT // 2:] << 16)
    yt_ref[...] = lax.bitcast_convert_type(packed, jnp.int32)
    z_ref[...] = lax.dot_general(xv, ws_ref[...], dn,
                                 preferred_element_type=jnp.float32) + b_ref[...]
    gi_ref[...] = nd_ref[...] * 4 + rel_ref[...]


def _combine_body(acc_ref, z_ref, o_ref):
    av = acc_ref[...]
    o_ref[...] = jnp.maximum(av[0] + av[1] + z_ref[...], 0.0)


def _make_sc_agg(N, OUT, E):
    NW = NC * NS
    EW = E // NW
    CH = 40
    NCH = EW // CH
    P = 4
    K = 2
    G = 40
    NG = N // G
    NGT = (NG + NS - 1) // NS
    GO = 80
    NGO = N // GO
    NGOT = (NGO + NS - 1) // NS
    assert EW * NW == E and NCH * CH == EW and NG * G == N
    NVR = OUT // 16

    mesh = plsc.VectorSubcoreMesh(core_axis_name="c", subcore_axis_name="s",
                                  num_cores=NC, num_subcores=NS)

    @functools.partial(
        pl.kernel,
        out_type=jax.ShapeDtypeStruct((NC, N, OUT), jnp.float32),
        mesh=mesh,
        compiler_params=pltpu.CompilerParams(needs_layout_passes=False,
                                             use_tc_tiling_on_sc=False),
        scratch_types=[
            pltpu.VMEM_SHARED((N, OUT), jnp.float32),
            pltpu.VMEM((EW,), jnp.int32),
            pltpu.VMEM((EW,), jnp.float32),
            [pltpu.VMEM((CH,), jnp.int32) for _ in range(P)],
            [pltpu.VMEM((CH, OUT // 2), jnp.int32) for _ in range(P)],
            [pltpu.VMEM((CH, OUT), jnp.float32) for _ in range(2)],
            pltpu.VMEM((G, OUT), jnp.float32),
            [pltpu.SemaphoreType.DMA for _ in range(P)],
            [pltpu.SemaphoreType.DMA for _ in range(2)],
            [pltpu.SemaphoreType.DMA for _ in range(P)],
        ],
    )
    def sc_agg(yt_hbm, gi_hbm, ni_hbm, w_hbm, acc_out,
               acc_sp, gi_v, w_v, ni, rows_bf, rows, zbuf, sem_g, sem_s, sem_n):
        cid = lax.axis_index("c")
        sid = lax.axis_index("s")
        wid = cid * NS + sid
        base = wid * EW

        zero16 = jnp.zeros((16,), jnp.float32)

        def zb_body(r, _):
            for j in range(NVR):
                zbuf[r, pl.ds(j * 16, 16)] = zero16
            return 0
        lax.fori_loop(0, G, zb_body, 0)

        pltpu.sync_copy(gi_hbm.at[pl.ds(base, EW)], gi_v)
        pltpu.sync_copy(w_hbm.at[pl.ds(base, EW)], w_v)

        def zero_body(t, _):
            g = sid + t * NS

            @pl.when(g < NG)
            def _():
                pltpu.sync_copy(zbuf, acc_sp.at[pl.ds(g * G, G)])
            return 0
        lax.fori_loop(0, NGT, zero_body, 0)
        plsc.subcore_barrier()

        def issue_front(c, b):
            pltpu.async_copy(yt_hbm.at[gi_v.at[pl.ds(c * CH, CH)]],
                             rows_bf[b], sem_g[b])
            pltpu.async_copy(ni_hbm.at[pl.ds(base + c * CH, CH)],
                             ni[b], sem_n[b])

        def wait_front(c, b):
            pltpu.make_async_copy(yt_hbm.at[gi_v.at[pl.ds(c * CH, CH)]],
                                  rows_bf[b], sem_g[b]).wait()
            pltpu.make_async_copy(ni_hbm.at[pl.ds(base + c * CH, CH)],
                                  ni[b], sem_n[b]).wait()

        def issue_scatter(c, b):
            pltpu.async_copy(rows[b % 2], acc_sp.at[ni[b]], sem_s[b % 2],
                             add=True)

        def wait_scatter(c, b):
            pltpu.make_async_copy(rows[b % 2], acc_sp.at[ni[b]],
                                  sem_s[b % 2]).wait()

        def compute(c, b):
            rbf = rows_bf[b]
            rf = rows[b % 2]
            for off, k0 in ((0, 0), (16, 0), (24, 8)):
                w16 = w_v[pl.ds(c * CH + off, 16)]
                for k in range(k0, 16):
                    i = off + k
                    wgt = w16[k]
                    for j in range(OUT // 32):
                        v16 = rbf[i, pl.ds(j * 16, 16)]
                        lo, hi = plsc.unpack(
                            plsc.bitcast(v16, jnp.bfloat16),
                            format=plsc.PackFormat.INTERLEAVED)
                        rf[i, pl.ds(j * 32, 16)] = lo * wgt
                        rf[i, pl.ds(j * 32 + 16, 16)] = hi * wgt

        for b in range(K):
            issue_front(b, b)

        def quad_body(q, _):
            for b in range(P):
                c = q * P + b

                @pl.when(c < NCH)
                def _slot():
                    @pl.when(c >= K)
                    def _():
                        wait_scatter(c - K, (b - K) % P)

                    @pl.when(c + K < NCH)
                    def _():
                        issue_front(c + K, (b + K) % P)
                    wait_front(c, b)
                    compute(c, b)
                    issue_scatter(c, b)
            return 0
        lax.fori_loop(0, (NCH + P - 1) // P, quad_body, 0)

        for t in range(K):
            c = NCH - K + t
            wait_scatter(c, c % P)

        plsc.subcore_barrier()

        def out_body(t, _):
            g = sid + t * NS

            @pl.when(g < NGO)
            def _():
                pltpu.sync_copy(acc_sp.at[pl.ds(g * GO, GO)],
                                acc_out.at[cid, pl.ds(g * GO, GO)])
            return 0
        lax.fori_loop(0, NGOT, out_body, 0)

    return sc_agg


def kernel(x, edge_index, relation, edge_weight, ccle, W_lin, b_lin,
           W_self, b_self):
    N, D = x.shape
    OUT = W_lin.shape[0]
    E = edge_weight.shape[0]
    R = ccle.shape[0]

    H = OUT // 2
    uperm = jnp.array(
        [32 * (c // 16) + (c % 16) if c < H
         else 32 * ((c - H) // 16) + 16 + ((c - H) % 16)
         for c in range(OUT)], dtype=jnp.int32)
    W_x = W_lin[:, :D][uperm]
    W_c = W_lin[:, D:][uperm]
    bias = (b_lin + b_self).reshape(1, OUT)
    node_in = edge_index[0]
    node_out = edge_index[1]

    BP = 1000
    NB = N // BP
    EL = 400
    EB = E // NB // EL
    yt, z, gidx = pl.pallas_call(
        _pre_body,
        grid=(NB,),
        in_specs=[
            pl.BlockSpec((BP, D), lambda i: (i, 0)),
            pl.BlockSpec((OUT, D), lambda i: (0, 0)),
            pl.BlockSpec((OUT, D), lambda i: (0, 0)),
            pl.BlockSpec((1, OUT), lambda i: (0, 0)),
            pl.BlockSpec((R, ccle.shape[1]), lambda i: (0, 0)),
            pl.BlockSpec((OUT, ccle.shape[1]), lambda i: (0, 0)),
            pl.BlockSpec((EB, EL), lambda i: (i, 0)),
            pl.BlockSpec((EB, EL), lambda i: (i, 0)),
        ],
        out_specs=[
            pl.BlockSpec((BP * R, OUT // 2), lambda i: (i, 0)),
            pl.BlockSpec((BP, OUT), lambda i: (i, 0)),
            pl.BlockSpec((EB, EL), lambda i: (i, 0)),
        ],
        out_shape=[
            jax.ShapeDtypeStruct((N * R, OUT // 2), jnp.int32),
            jax.ShapeDtypeStruct((N, OUT), jnp.float32),
            jax.ShapeDtypeStruct((NB * EB, EL), jnp.int32),
        ],
    )(x, W_x, W_self, bias, ccle, W_c,
      node_out.reshape(NB * EB, EL), relation.reshape(NB * EB, EL))

    acc = _make_sc_agg(N, OUT, E)(yt, gidx.reshape(E), node_in, edge_weight)

    BC = 1000
    out = pl.pallas_call(
        _combine_body,
        grid=(N // BC,),
        in_specs=[
            pl.BlockSpec((NC, BC, OUT), lambda i: (0, i, 0)),
            pl.BlockSpec((BC, OUT), lambda i: (i, 0)),
        ],
        out_specs=pl.BlockSpec((BC, OUT), lambda i: (i, 0)),
        out_shape=jax.ShapeDtypeStruct((N, OUT), jnp.float32),
    )(acc, z)
    return out


# speedup vs baseline: 1.1469x; 1.1469x over previous
import functools

import jax
import jax.numpy as jnp
from jax import lax
from jax.experimental import pallas as pl
from jax.experimental.pallas import tpu as pltpu
from jax.experimental.pallas import tpu_sc as plsc

NC = 2
NS = 16


def _pre_body(x_ref, wx_ref, ws_ref, b_ref, ccle_ref, wc_ref, nd_ref, rel_ref,
              yt_ref, z_ref, gi_ref):
    dn = (((1,), (1,)), ((), ()))
    xv = x_ref[...]
    y = lax.dot_general(xv, wx_ref[...], dn,
                        preferred_element_type=jnp.float32)
    t4 = lax.dot_general(ccle_ref[...], wc_ref[...], dn,
                         preferred_element_type=jnp.float32)
    B, OUT = y.shape
    R = t4.shape[0]
    yt = (y[:, None, :] + t4[None, :, :]).reshape(B * R, OUT)
    bits = lax.bitcast_convert_type(yt, jnp.uint32)
    rnd = (bits + 0x7FFF + ((bits >> 16) & 1)) >> 16
    packed = rnd[:, : OUT // 2] | (rnd[:, OUT // 2:] << 16)
    yt_ref[...] = lax.bitcast_convert_type(packed, jnp.int32)
    z_ref[...] = lax.dot_general(xv, ws_ref[...], dn,
                                 preferred_element_type=jnp.float32) + b_ref[...]
    gi_ref[...] = nd_ref[...] * 4 + rel_ref[...]


def _combine_body(acc_ref, z_ref, o_ref):
    av = acc_ref[...]
    o_ref[...] = jnp.maximum(av[0] + av[1] + z_ref[...], 0.0)


def _make_sc_agg(N, OUT, E):
    NW = NC * NS
    EW = E // NW
    CH = 40
    NCH = EW // CH
    P = 4
    K = 2
    G = 40
    NG = N // G
    NGT = (NG + NS - 1) // NS
    assert EW * NW == E and NCH * CH == EW and NG * G == N
    NVR = OUT // 16

    mesh = plsc.VectorSubcoreMesh(core_axis_name="c", subcore_axis_name="s",
                                  num_cores=NC, num_subcores=NS)

    @functools.partial(
        pl.kernel,
        out_type=jax.ShapeDtypeStruct((NC, N, OUT), jnp.float32),
        mesh=mesh,
        compiler_params=pltpu.CompilerParams(needs_layout_passes=False,
                                             use_tc_tiling_on_sc=False),
        scratch_types=[
            pltpu.VMEM_SHARED((N, OUT), jnp.float32),
            pltpu.VMEM((EW,), jnp.int32),
            pltpu.VMEM((EW,), jnp.float32),
            [pltpu.VMEM((CH,), jnp.int32) for _ in range(P)],
            [pltpu.VMEM((CH, OUT // 2), jnp.int32) for _ in range(P)],
            [pltpu.VMEM((CH, OUT), jnp.float32) for _ in range(2)],
            pltpu.VMEM((G, OUT), jnp.float32),
            [pltpu.SemaphoreType.DMA for _ in range(P)],
            [pltpu.SemaphoreType.DMA for _ in range(2)],
            [pltpu.SemaphoreType.DMA for _ in range(P)],
        ],
    )
    def sc_agg(yt_hbm, gi_hbm, ni_hbm, w_hbm, acc_out,
               acc_sp, gi_v, w_v, ni, rows_bf, rows, zbuf, sem_g, sem_s, sem_n):
        cid = lax.axis_index("c")
        sid = lax.axis_index("s")
        wid = cid * NS + sid
        base = wid * EW

        zero16 = jnp.zeros((16,), jnp.float32)

        def zb_body(r, _):
            for j in range(NVR):
                zbuf[r, pl.ds(j * 16, 16)] = zero16
            return 0
        lax.fori_loop(0, G, zb_body, 0)

        pltpu.sync_copy(gi_hbm.at[pl.ds(base, EW)], gi_v)
        pltpu.sync_copy(w_hbm.at[pl.ds(base, EW)], w_v)

        def zero_body(t, _):
            g = sid + t * NS

            @pl.when(g < NG)
            def _():
                pltpu.sync_copy(zbuf, acc_sp.at[pl.ds(g * G, G)])
            return 0
        lax.fori_loop(0, NGT, zero_body, 0)
        plsc.subcore_barrier()

        def issue_front(c, b):
            pltpu.async_copy(yt_hbm.at[gi_v.at[pl.ds(c * CH, CH)]],
                             rows_bf[b], sem_g[b])
            pltpu.async_copy(ni_hbm.at[pl.ds(base + c * CH, CH)],
                             ni[b], sem_n[b])

        def wait_front(c, b):
            pltpu.make_async_copy(yt_hbm.at[gi_v.at[pl.ds(c * CH, CH)]],
                                  rows_bf[b], sem_g[b]).wait()
            pltpu.make_async_copy(ni_hbm.at[pl.ds(base + c * CH, CH)],
                                  ni[b], sem_n[b]).wait()

        def issue_scatter(c, b):
            pltpu.async_copy(rows[b % 2], acc_sp.at[ni[b]], sem_s[b % 2],
                             add=True)

        def wait_scatter(c, b):
            pltpu.make_async_copy(rows[b % 2], acc_sp.at[ni[b]],
                                  sem_s[b % 2]).wait()

        def compute(c, b):
            rbf = rows_bf[b]
            rf = rows[b % 2]
            for off, k0 in ((0, 0), (16, 0), (24, 8)):
                w16 = w_v[pl.ds(c * CH + off, 16)]
                for k in range(k0, 16):
                    i = off + k
                    wgt = w16[k]
                    for j in range(OUT // 32):
                        v16 = rbf[i, pl.ds(j * 16, 16)]
                        lo, hi = plsc.unpack(
                            plsc.bitcast(v16, jnp.bfloat16),
                            format=plsc.PackFormat.INTERLEAVED)
                        rf[i, pl.ds(j * 32, 16)] = lo * wgt
                        rf[i, pl.ds(j * 32 + 16, 16)] = hi * wgt

        for b in range(K):
            issue_front(b, b)

        def quad_body(q, _):
            for b in range(P):
                c = q * P + b

                @pl.when(c < NCH)
                def _slot():
                    wait_front(c, b)

                    @pl.when(c >= K)
                    def _():
                        wait_scatter(c - K, (b - K) % P)

                    @pl.when(c + K < NCH)
                    def _():
                        issue_front(c + K, (b + K) % P)
                    compute(c, b)
                    issue_scatter(c, b)
            return 0
        lax.fori_loop(0, (NCH + P - 1) // P, quad_body, 0)

        for t in range(K):
            c = NCH - K + t
            wait_scatter(c, c % P)

        plsc.subcore_barrier()

        def out_body(t, _):
            g = sid + t * NS

            @pl.when(g < NG)
            def _():
                pltpu.sync_copy(acc_sp.at[pl.ds(g * G, G)],
                                acc_out.at[cid, pl.ds(g * G, G)])
            return 0
        lax.fori_loop(0, NGT, out_body, 0)

    return sc_agg


def kernel(x, edge_index, relation, edge_weight, ccle, W_lin, b_lin,
           W_self, b_self):
    N, D = x.shape
    OUT = W_lin.shape[0]
    E = edge_weight.shape[0]
    R = ccle.shape[0]

    H = OUT // 2
    uperm = jnp.array(
        [32 * (c // 16) + (c % 16) if c < H
         else 32 * ((c - H) // 16) + 16 + ((c - H) % 16)
         for c in range(OUT)], dtype=jnp.int32)
    W_x = W_lin[:, :D][uperm]
    W_c = W_lin[:, D:][uperm]
    bias = (b_lin + b_self).reshape(1, OUT)
    node_in = edge_index[0]
    node_out = edge_index[1]

    BP = 1000
    NB = N // BP
    EL = 400
    EB = E // NB // EL
    yt, z, gidx = pl.pallas_call(
        _pre_body,
        grid=(NB,),
        in_specs=[
            pl.BlockSpec((BP, D), lambda i: (i, 0)),
            pl.BlockSpec((OUT, D), lambda i: (0, 0)),
            pl.BlockSpec((OUT, D), lambda i: (0, 0)),
            pl.BlockSpec((1, OUT), lambda i: (0, 0)),
            pl.BlockSpec((R, ccle.shape[1]), lambda i: (0, 0)),
            pl.BlockSpec((OUT, ccle.shape[1]), lambda i: (0, 0)),
            pl.BlockSpec((EB, EL), lambda i: (i, 0)),
            pl.BlockSpec((EB, EL), lambda i: (i, 0)),
        ],
        out_specs=[
            pl.BlockSpec((BP * R, OUT // 2), lambda i: (i, 0)),
            pl.BlockSpec((BP, OUT), lambda i: (i, 0)),
            pl.BlockSpec((EB, EL), lambda i: (i, 0)),
        ],
        out_shape=[
            jax.ShapeDtypeStruct((N * R, OUT // 2), jnp.int32),
            jax.ShapeDtypeStruct((N, OUT), jnp.float32),
            jax.ShapeDtypeStruct((NB * EB, EL), jnp.int32),
        ],
    )(x, W_x, W_self, bias, ccle, W_c,
      node_out.reshape(NB * EB, EL), relation.reshape(NB * EB, EL))

    acc = _make_sc_agg(N, OUT, E)(yt, gidx.reshape(E), node_in, edge_weight)

    BC = 1000
    out = pl.pallas_call(
        _combine_body,
        grid=(N // BC,),
        in_specs=[
            pl.BlockSpec((NC, BC, OUT), lambda i: (0, i, 0)),
            pl.BlockSpec((BC, OUT), lambda i: (i, 0)),
        ],
        out_specs=pl.BlockSpec((BC, OUT), lambda i: (i, 0)),
        out_shape=jax.ShapeDtypeStruct((N, OUT), jnp.float32),
    )(acc, z)
    return out
